# Initial kernel scaffold; baseline (speedup 1.0000x reference)
import jax
import jax.numpy as jnp
from jax.experimental import pallas as pl


def kernel(x, scale, bias, binary):
    raise NotImplementedError("write your pallas kernel here")


# speedup vs baseline: 369.0416x; 369.0416x over previous
import jax
import jax.numpy as jnp
from jax import lax
from jax.experimental import pallas as pl
from jax.experimental.pallas import tpu as pltpu

NX = 768
NF = 768
NX8 = NX // 8
NBITS = 8


def _body(x_ref, scale_ref, bias_ref, binary_ref, out_ref):
    g_row = lax.broadcasted_iota(jnp.int32, (NX8, NX), 0)
    c_col = lax.broadcasted_iota(jnp.int32, (NX8, NX), 1)
    expand = jnp.where(c_col // 8 == g_row, 1.0, 0.0).astype(jnp.float32)

    col = lax.broadcasted_iota(jnp.int32, (NF, NX), 1)
    shift = 7 - (col % 8)

    wq = jnp.zeros((NF, NX), jnp.float32)
    for i in range(NBITS):
        codes = binary_ref[i].astype(jnp.float32)
        codes_exp = lax.dot_general(
            codes, expand, (((1,), (0,)), ((), ())),
            preferred_element_type=jnp.float32,
        )
        bits = (codes_exp.astype(jnp.int32) >> shift) & 1
        signs = (2 * bits - 1).astype(jnp.float32)
        wq = wq + scale_ref[i] * signs

    out = lax.dot_general(
        x_ref[...], wq, (((1,), (1,)), ((), ())),
        preferred_element_type=jnp.float32,
    )
    out_ref[...] = out + bias_ref[...]


def kernel(x, scale, bias, binary):
    size_out = x.shape[:-1] + (NF,)
    x2 = x.reshape(-1, NX)
    out = pl.pallas_call(
        _body,
        out_shape=jax.ShapeDtypeStruct((x2.shape[0], NF), jnp.float32),
    )(x2, scale, bias.reshape(1, NF), binary)
    return out.reshape(size_out)
